# Initial kernel scaffold; baseline (speedup 1.0000x reference)
import jax
import jax.numpy as jnp
from jax.experimental import pallas as pl


def kernel(inputs, table, W, b):
    raise NotImplementedError("write your pallas kernel here")


# speedup vs baseline: 1.6858x; 1.6858x over previous
import functools

import jax
import jax.numpy as jnp
from jax import lax
from jax.experimental import pallas as pl
from jax.experimental.pallas import tpu as pltpu
from jax.experimental.pallas import tpu_sc as plsc

_VOCAB = 1000000
_EMBED = 32
_BATCH = 16384
_CTX = 20

_NUM_TILES = 32
_CHUNK_B = 64
_CHUNK_IDX = _CHUNK_B * _CTX
_GATHER_W = 128
_N_GATHERS = _CHUNK_IDX // _GATHER_W


def _sc_pool(table, idx_flat):
  mesh = plsc.VectorSubcoreMesh(core_axis_name="c", subcore_axis_name="s")
  n_chunks = _BATCH // _CHUNK_B

  @functools.partial(
      pl.kernel,
      out_type=jax.ShapeDtypeStruct((_BATCH, _EMBED), jnp.float32),
      mesh=mesh,
      compiler_params=pltpu.CompilerParams(use_tc_tiling_on_sc=False),
      scratch_types=[
          pltpu.VMEM((_CHUNK_IDX, _EMBED), jnp.float32),
          pltpu.SemaphoreType.DMA,
      ],
  )
  def pool_kernel(table_hbm, idx_hbm, out_hbm, rows_v, sem):
    def body(idx_v, out_v):
      copies = []
      for k in range(_N_GATHERS):
        copies.append(
            pltpu.async_copy(
                table_hbm.at[idx_v.at[0, pl.ds(k * _GATHER_W, _GATHER_W)]],
                rows_v.at[pl.ds(k * _GATHER_W, _GATHER_W)],
                sem,
            )
        )
      for c in copies:
        c.wait()

      @pl.loop(0, _CHUNK_B)
      def _(b):
        base = b * _CTX
        s0 = rows_v[base, pl.ds(0, 16)]
        s1 = rows_v[base, pl.ds(16, 16)]
        for j in range(1, _CTX):
          s0 += rows_v[base + j, pl.ds(0, 16)]
          s1 += rows_v[base + j, pl.ds(16, 16)]
        out_v[b, pl.ds(0, 16)] = s0
        out_v[b, pl.ds(16, 16)] = s1

    pltpu.emit_pipeline(
        body,
        grid=(n_chunks,),
        in_specs=[
            pl.BlockSpec((1, _CHUNK_IDX), index_map=lambda i: (0, i)),
        ],
        out_specs=[
            pl.BlockSpec((_CHUNK_B, _EMBED), index_map=lambda i: (i, 0)),
        ],
        core_axis_name=("c", "s"),
        dimension_semantics=(pltpu.PARALLEL,),
    )(idx_hbm, out_hbm)

  return pool_kernel(table, idx_flat)


def _tc_project(pooled, W, b):
  def proj_kernel(pooled_ref, w_ref, b_ref, out_ref):
    w_row = w_ref[...]
    prod = pooled_ref[...] * w_row
    s = jnp.sum(prod, axis=1, keepdims=True)
    out_ref[...] = s * (1.0 / _CTX) + b_ref[0, 0]

  return pl.pallas_call(
      proj_kernel,
      out_shape=jax.ShapeDtypeStruct((_BATCH, 1), jnp.float32),
      in_specs=[
          pl.BlockSpec(memory_space=pltpu.VMEM),
          pl.BlockSpec(memory_space=pltpu.VMEM),
          pl.BlockSpec(memory_space=pltpu.SMEM),
      ],
      out_specs=pl.BlockSpec(memory_space=pltpu.VMEM),
  )(pooled, W, b.reshape(1, 1))


@jax.jit
def kernel(inputs, table, W, b):
  idx_flat = inputs.reshape(1, _BATCH * _CTX)
  pooled = _sc_pool(table, idx_flat)
  return _tc_project(pooled, W, b)
